# baseline (speedup 1.0000x reference)
import functools

import jax
import jax.numpy as jnp
from jax import lax
from jax.experimental import pallas as pl
from jax.experimental.pallas import tpu as pltpu
from jax.experimental.pallas import tpu_sc as plsc

N = 10000
E = 320000
H = 128
NORM = 100.0

NC = 2
NS = 16
CH = 80

BE = 2560
RB = BE // 512
GRID = E // BE
GROWS = E // 512

ES = E // NS
NCHS = ES // CH
SLICES = (42, 42, 41)
RPT = 640


def _pre_node(h, W1a, W1b):
    def body(h_ref, wa_ref, wb_ref, a_ref, b_ref):
        hv = h_ref[...]
        a_ref[...] = jnp.dot(hv, wa_ref[...], preferred_element_type=jnp.float32)
        b_ref[...] = jnp.dot(hv, wb_ref[...], preferred_element_type=jnp.float32)

    BN = 2000
    return pl.pallas_call(
        body,
        grid=(N // BN,),
        in_specs=[
            pl.BlockSpec((BN, H), lambda i: (i, 0)),
            pl.BlockSpec((H, H), lambda i: (0, 0)),
            pl.BlockSpec((H, H), lambda i: (0, 0)),
        ],
        out_specs=[
            pl.BlockSpec((BN, H), lambda i: (i, 0)),
            pl.BlockSpec((BN, H), lambda i: (i, 0)),
        ],
        out_shape=[
            jax.ShapeDtypeStruct((N, H), jnp.float32),
            jax.ShapeDtypeStruct((N, H), jnp.float32),
        ],
    )(h, W1a, W1b)


def _sc_gather_slice(A, B, idx3r_s, idx3c_s, nch):
    mesh = plsc.VectorSubcoreMesh(core_axis_name="c", subcore_axis_name="s")
    es_tile = nch * CH
    es_slice = es_tile * NS

    @functools.partial(
        pl.kernel,
        mesh=mesh,
        out_type=[
            jax.ShapeDtypeStruct((es_slice, H), jnp.float32),
            jax.ShapeDtypeStruct((es_slice, H), jnp.float32),
        ],
        scratch_types=[
            pltpu.VMEM((nch, CH), jnp.int32),
            pltpu.VMEM((CH, H), jnp.float32),
            pltpu.VMEM((CH, H), jnp.float32),
            pltpu.VMEM_SHARED((N, H), jnp.float32),
            pltpu.SemaphoreType.DMA,
            pltpu.SemaphoreType.DMA,
        ],
    )
    def k(a_hbm, b_hbm, ir_hbm, ic_hbm, p_hbm, q_hbm, ix_v, buf0, buf1,
          table, sem0, sem1):
        cid = lax.axis_index("c")
        sid = lax.axis_index("s")

        def run(t_hbm, i_hbm, o_hbm):
            row0 = sid * RPT

            @pl.when(sid < NS - 1)
            def _():
                pltpu.sync_copy(t_hbm.at[pl.ds(row0, RPT)],
                                table.at[pl.ds(row0, RPT)])

            @pl.when(sid == NS - 1)
            def _():
                last = N - (NS - 1) * RPT
                pltpu.sync_copy(t_hbm.at[pl.ds(row0, last)],
                                table.at[pl.ds(row0, last)])

            pltpu.sync_copy(i_hbm.at[sid], ix_v)
            plsc.subcore_barrier()

            base = sid * es_tile
            bufs = (buf0, buf1)
            sems = (sem0, sem1)
            pltpu.async_copy(table.at[ix_v.at[0]], buf0, sem0)

            def body(jj, carry):
                for k2 in range(2):
                    j = jj * 2 + k2
                    cur = k2
                    nxt = 1 - k2
                    jn = j + 1

                    @pl.when(jn < nch)
                    def _():
                        pltpu.async_copy(table.at[ix_v.at[jn]],
                                         bufs[nxt], sems[nxt])

                    off = base + j * CH
                    pltpu.make_async_copy(table.at[ix_v.at[j]],
                                          bufs[cur], sems[cur]).wait()
                    pltpu.sync_copy(bufs[cur], o_hbm.at[pl.ds(off, CH)])
                return carry

            lax.fori_loop(0, nch // 2, body, 0)

        @pl.when(cid == 0)
        def _():
            run(a_hbm, ir_hbm, p_hbm)

        @pl.when(cid == 1)
        def _():
            run(b_hbm, ic_hbm, q_hbm)

    return k(A, B, idx3r_s, idx3c_s)


def _edge_mlp(P, Q, cd, ea, w1c, b1r, W2, b2r, W3, I512, units, uoff):
    def body(p_ref, q_ref, cd_ref, ea_ref, w1c_ref,
             b1_ref, w2_ref, b2_ref, w3_ref, i_ref,
             tx_ref, ty_ref, tz_ref):
        ident = i_ref[...]
        eaw = jnp.dot(ea_ref[...], w1c_ref[...],
                      preferred_element_type=jnp.float32)

        s = p_ref[...] + q_ref[...] + eaw + b1_ref[...]
        x1 = jax.nn.silu(s)
        y = (jnp.dot(x1, w2_ref[...], preferred_element_type=jnp.float32)
             + b2_ref[...])
        x2 = jax.nn.silu(y)
        m_col = jnp.dot(x2, w3_ref[...], preferred_element_type=jnp.float32)

        dn_row = (((0,), (0,)), ((), ()))
        m_rows = jnp.concatenate(
            [lax.dot_general(m_col[r * 512:(r + 1) * 512, :], ident, dn_row,
                             preferred_element_type=jnp.float32)
             for r in range(RB)], axis=0)

        cd_full = cd_ref[...]
        cds = [lax.dot_general(cd_full[r * 512:(r + 1) * 512, :], ident,
                               dn_row, preferred_element_type=jnp.float32)
               for r in range(RB)]
        cx_rows = jnp.concatenate([c[0:1] for c in cds], axis=0)
        cy_rows = jnp.concatenate([c[1:2] for c in cds], axis=0)
        cz_rows = jnp.concatenate([c[2:3] for c in cds], axis=0)

        tx_ref[...] = (cx_rows * m_rows)[None]
        ty_ref[...] = (cy_rows * m_rows)[None]
        tz_ref[...] = (cz_rows * m_rows)[None]

    rspec = pl.BlockSpec((1, RB, 512), lambda i: (i, 0, 0))
    rshape = jax.ShapeDtypeStruct((units, RB, 512), jnp.float32)
    return pl.pallas_call(
        body,
        grid=(units,),
        in_specs=[
            pl.BlockSpec((BE, H), lambda i: (i, 0)),
            pl.BlockSpec((BE, H), lambda i: (i, 0)),
            pl.BlockSpec((BE, 3), lambda i: (i + uoff, 0)),
            pl.BlockSpec((BE, 1), lambda i: (i + uoff, 0)),
            pl.BlockSpec((1, H), lambda i: (0, 0)),
            pl.BlockSpec((1, H), lambda i: (0, 0)),
            pl.BlockSpec((H, H), lambda i: (0, 0)),
            pl.BlockSpec((1, H), lambda i: (0, 0)),
            pl.BlockSpec((H, 1), lambda i: (0, 0)),
            pl.BlockSpec((512, 512), lambda i: (0, 0)),
        ],
        out_specs=[rspec, rspec, rspec],
        out_shape=[rshape, rshape, rshape],
    )(P, Q, cd, ea, w1c, b1r, W2, b2r, W3, I512)


def _sc_scatter_finalize(tx1, ty1, tz1, idx3s, cx, cy, cz):
    mesh = plsc.VectorSubcoreMesh(core_axis_name="c", subcore_axis_name="s")

    @functools.partial(
        pl.kernel,
        mesh=mesh,
        out_type=[
            jax.ShapeDtypeStruct((N,), jnp.float32),
            jax.ShapeDtypeStruct((N,), jnp.float32),
            jax.ShapeDtypeStruct((N,), jnp.float32),
        ],
        scratch_types=[
            pltpu.VMEM((NCHS, CH), jnp.int32),
            pltpu.VMEM((ES,), jnp.float32),
            pltpu.VMEM((ES,), jnp.float32),
            pltpu.VMEM((ES,), jnp.float32),
            pltpu.VMEM((RPT,), jnp.float32),
            pltpu.VMEM((RPT,), jnp.float32),
            pltpu.VMEM((RPT,), jnp.float32),
            pltpu.VMEM_SHARED((N,), jnp.float32),
            pltpu.VMEM_SHARED((N,), jnp.float32),
            pltpu.VMEM_SHARED((N,), jnp.float32),
        ],
    )
    def k(tx_h, ty_h, tz_h, ix_h, cx_h, cy_h, cz_h, ox_h, oy_h, oz_h,
          ix_v, txv, tyv, tzv, avbuf, cbuf, obuf, accx, accy, accz):
        sid = lax.axis_index("s")
        base = sid * ES
        pltpu.sync_copy(ix_h.at[sid], ix_v)
        pltpu.sync_copy(tx_h.at[pl.ds(base, ES)], txv)
        pltpu.sync_copy(ty_h.at[pl.ds(base, ES)], tyv)
        pltpu.sync_copy(tz_h.at[pl.ds(base, ES)], tzv)

        def zb(i, carry):
            avbuf[pl.ds(i * 16, 16)] = jnp.zeros((16,), jnp.float32)
            return carry

        lax.fori_loop(0, RPT // 16, zb, 0)
        row0 = sid * RPT

        def zero_acc(nrows):
            pltpu.sync_copy(avbuf.at[pl.ds(0, nrows)], accx.at[pl.ds(row0, nrows)])
            pltpu.sync_copy(avbuf.at[pl.ds(0, nrows)], accy.at[pl.ds(row0, nrows)])
            pltpu.sync_copy(avbuf.at[pl.ds(0, nrows)], accz.at[pl.ds(row0, nrows)])

        @pl.when(sid < NS - 1)
        def _():
            zero_acc(RPT)

        @pl.when(sid == NS - 1)
        def _():
            zero_acc(N - (NS - 1) * RPT)

        plsc.subcore_barrier()

        def body(j, carry):
            src = pl.ds(j * CH, CH)
            ixr = ix_v.at[j]
            pltpu.sync_copy(txv.at[src], accx.at[ixr], add=True)
            pltpu.sync_copy(tyv.at[src], accy.at[ixr], add=True)
            pltpu.sync_copy(tzv.at[src], accz.at[ixr], add=True)
            return carry

        lax.fori_loop(0, NCHS, body, 0)
        plsc.subcore_barrier()

        def fin(acc, c_h, o_h, nrows):
            pltpu.sync_copy(acc.at[pl.ds(row0, nrows)], avbuf.at[pl.ds(0, nrows)])
            pltpu.sync_copy(c_h.at[pl.ds(row0, nrows)], cbuf.at[pl.ds(0, nrows)])

            def fb(i, carry):
                sl = pl.ds(i * 16, 16)
                obuf[sl] = cbuf[sl] + avbuf[sl] * (1.0 / NORM)
                return carry

            lax.fori_loop(0, nrows // 16, fb, 0)
            pltpu.sync_copy(obuf.at[pl.ds(0, nrows)], o_h.at[pl.ds(row0, nrows)])

        def fin_all(nrows):
            fin(accx, cx_h, ox_h, nrows)
            fin(accy, cy_h, oy_h, nrows)
            fin(accz, cz_h, oz_h, nrows)

        @pl.when(sid < NS - 1)
        def _():
            fin_all(RPT)

        @pl.when(sid == NS - 1)
        def _():
            fin_all(N - (NS - 1) * RPT)

    return k(tx1, ty1, tz1, idx3s, cx, cy, cz)


def kernel(h, coord, edge_index, coord_diff, edge_attr, W1, b1, W2, b2, W3):
    f32 = jnp.float32
    row = edge_index[0].astype(jnp.int32)
    col = edge_index[1].astype(jnp.int32)

    W1a = W1[:H]
    W1b = W1[H:2 * H]
    w1c = W1[2 * H:2 * H + 1]
    I512 = jnp.eye(512, dtype=f32)

    A, B = _pre_node(h, W1a, W1b)

    b1r = b1.reshape(1, H)
    b2r = b2.reshape(1, H)
    txs, tys, tzs = [], [], []
    e0 = 0
    u0 = 0
    for units_s in SLICES:
        es = units_s * BE
        nch = es // (NS * CH)
        rows_s = lax.slice(row, (e0,), (e0 + es,)).reshape(NS, nch, CH)
        cols_s = lax.slice(col, (e0,), (e0 + es,)).reshape(NS, nch, CH)
        Pg, Qg = _sc_gather_slice(A, B, rows_s, cols_s, nch)
        tx3, ty3, tz3 = _edge_mlp(Pg, Qg, coord_diff, edge_attr,
                                  w1c, b1r, W2, b2r, W3, I512, units_s, u0)
        txs.append(tx3.reshape(es))
        tys.append(ty3.reshape(es))
        tzs.append(tz3.reshape(es))
        e0 += es
        u0 += units_s

    tx1 = jnp.concatenate(txs)
    ty1 = jnp.concatenate(tys)
    tz1 = jnp.concatenate(tzs)
    idx3s = row.reshape(NS, NCHS, CH)
    ox, oy, oz = _sc_scatter_finalize(tx1, ty1, tz1, idx3s,
                                      coord[:, 0], coord[:, 1], coord[:, 2])
    return jnp.stack([ox, oy, oz], axis=1)


# speedup vs baseline: 1.0671x; 1.0671x over previous
import functools

import jax
import jax.numpy as jnp
from jax import lax
from jax.experimental import pallas as pl
from jax.experimental.pallas import tpu as pltpu
from jax.experimental.pallas import tpu_sc as plsc

N = 10000
E = 320000
H = 128
NORM = 100.0

NC = 2
NS = 16
CH = 80

BE = 2560
RB = BE // 512
GRID = E // BE
GROWS = E // 512

ES = E // NS
NCHS = ES // CH
ES2 = E // 32
NCH2 = ES2 // CH
NP = 10240
RPT = 640


def _pre_node(h, W1a, W1b):
    def body(h_ref, wa_ref, wb_ref, a_ref, b_ref):
        hv = h_ref[...]
        a_ref[...] = jnp.dot(hv, wa_ref[...], preferred_element_type=jnp.float32)
        b_ref[...] = jnp.dot(hv, wb_ref[...], preferred_element_type=jnp.float32)

    BN = 2000
    return pl.pallas_call(
        body,
        grid=(N // BN,),
        in_specs=[
            pl.BlockSpec((BN, H), lambda i: (i, 0)),
            pl.BlockSpec((H, H), lambda i: (0, 0)),
            pl.BlockSpec((H, H), lambda i: (0, 0)),
        ],
        out_specs=[
            pl.BlockSpec((BN, H), lambda i: (i, 0)),
            pl.BlockSpec((BN, H), lambda i: (i, 0)),
        ],
        out_shape=[
            jax.ShapeDtypeStruct((N, H), jnp.float32),
            jax.ShapeDtypeStruct((N, H), jnp.float32),
        ],
    )(h, W1a, W1b)


def _sc_gather(A, B, idx4r, idx4c):
    mesh = plsc.VectorSubcoreMesh(core_axis_name="c", subcore_axis_name="s")
    NHF = NCHS // 2

    @functools.partial(
        pl.kernel,
        mesh=mesh,
        out_type=[
            jax.ShapeDtypeStruct((E, H), jnp.float32),
            jax.ShapeDtypeStruct((E, H), jnp.float32),
        ],
        scratch_types=[
            pltpu.VMEM((NHF, CH), jnp.int32),
            pltpu.VMEM((CH, H), jnp.float32),
            pltpu.VMEM((CH, H), jnp.float32),
            pltpu.VMEM_SHARED((N, H), jnp.float32),
            pltpu.SemaphoreType.DMA,
            pltpu.SemaphoreType.DMA,
        ],
    )
    def k(a_hbm, b_hbm, ir_hbm, ic_hbm, p_hbm, q_hbm, ix_v, buf0, buf1,
          table, sem0, sem1):
        cid = lax.axis_index("c")
        sid = lax.axis_index("s")

        def run(t_hbm, i_hbm, o_hbm):
            row0 = sid * RPT

            @pl.when(sid < NS - 1)
            def _():
                pltpu.sync_copy(t_hbm.at[pl.ds(row0, RPT)],
                                table.at[pl.ds(row0, RPT)])

            @pl.when(sid == NS - 1)
            def _():
                last = N - (NS - 1) * RPT
                pltpu.sync_copy(t_hbm.at[pl.ds(row0, last)],
                                table.at[pl.ds(row0, last)])

            plsc.subcore_barrier()

            base = sid * ES
            bufs = (buf0, buf1)
            sems = (sem0, sem1)

            for hf in range(2):
                pltpu.sync_copy(i_hbm.at[sid, hf], ix_v)
                cb = hf * NHF
                pltpu.async_copy(table.at[ix_v.at[0]], buf0, sem0)

                def body(jj, carry):
                    for k2 in range(2):
                        j = jj * 2 + k2
                        cur = k2
                        nxt = 1 - k2
                        jn = j + 1

                        @pl.when(jn < NHF)
                        def _():
                            pltpu.async_copy(table.at[ix_v.at[jn]],
                                             bufs[nxt], sems[nxt])

                        off = base + (cb + j) * CH
                        pltpu.make_async_copy(table.at[ix_v.at[j]],
                                              bufs[cur], sems[cur]).wait()
                        pltpu.sync_copy(bufs[cur], o_hbm.at[pl.ds(off, CH)])
                    return carry

                lax.fori_loop(0, NHF // 2, body, 0)
                jl = NHF - 1
                off = base + (cb + jl) * CH
                pltpu.make_async_copy(table.at[ix_v.at[jl]], bufs[jl % 2],
                                      sems[jl % 2]).wait()
                pltpu.sync_copy(bufs[jl % 2], o_hbm.at[pl.ds(off, CH)])

        @pl.when(cid == 0)
        def _():
            run(a_hbm, ir_hbm, p_hbm)

        @pl.when(cid == 1)
        def _():
            run(b_hbm, ic_hbm, q_hbm)

    return k(A, B, idx4r, idx4c)


def _edge_mlp(P, Q, cd, ea, w1c, b1r, W2, b2r, W3, I512):
    def body(p_ref, q_ref, cd_ref, ea_ref, w1c_ref,
             b1_ref, w2_ref, b2_ref, w3_ref, i_ref,
             tx_ref, ty_ref, tz_ref):
        ident = i_ref[...]
        eaw = jnp.dot(ea_ref[...], w1c_ref[...],
                      preferred_element_type=jnp.float32)

        s = p_ref[...] + q_ref[...] + eaw + b1_ref[...]
        x1 = jax.nn.silu(s)
        y = (jnp.dot(x1, w2_ref[...], preferred_element_type=jnp.float32)
             + b2_ref[...])
        x2 = jax.nn.silu(y)
        m_col = jnp.dot(x2, w3_ref[...], preferred_element_type=jnp.float32)

        dn_row = (((0,), (0,)), ((), ()))
        m_rows = jnp.concatenate(
            [lax.dot_general(m_col[r * 512:(r + 1) * 512, :], ident, dn_row,
                             preferred_element_type=jnp.float32)
             for r in range(RB)], axis=0)

        cd_full = cd_ref[...]
        cds = [lax.dot_general(cd_full[r * 512:(r + 1) * 512, :], ident,
                               dn_row, preferred_element_type=jnp.float32)
               for r in range(RB)]
        cx_rows = jnp.concatenate([c[0:1] for c in cds], axis=0)
        cy_rows = jnp.concatenate([c[1:2] for c in cds], axis=0)
        cz_rows = jnp.concatenate([c[2:3] for c in cds], axis=0)

        tx_ref[...] = (cx_rows * m_rows)[None]
        ty_ref[...] = (cy_rows * m_rows)[None]
        tz_ref[...] = (cz_rows * m_rows)[None]

    rspec = pl.BlockSpec((1, RB, 512), lambda i: (i, 0, 0))
    rshape = jax.ShapeDtypeStruct((GRID, RB, 512), jnp.float32)
    return pl.pallas_call(
        body,
        grid=(GRID,),
        in_specs=[
            pl.BlockSpec((BE, H), lambda i: (i, 0)),
            pl.BlockSpec((BE, H), lambda i: (i, 0)),
            pl.BlockSpec((BE, 3), lambda i: (i, 0)),
            pl.BlockSpec((BE, 1), lambda i: (i, 0)),
            pl.BlockSpec((1, H), lambda i: (0, 0)),
            pl.BlockSpec((1, H), lambda i: (0, 0)),
            pl.BlockSpec((H, H), lambda i: (0, 0)),
            pl.BlockSpec((1, H), lambda i: (0, 0)),
            pl.BlockSpec((H, 1), lambda i: (0, 0)),
            pl.BlockSpec((512, 512), lambda i: (0, 0)),
        ],
        out_specs=[rspec, rspec, rspec],
        out_shape=[rshape, rshape, rshape],
    )(P, Q, cd, ea, w1c, b1r, W2, b2r, W3, I512)


def _sc_scatter(tx1, ty1, tz1, idx3s):
    mesh = plsc.VectorSubcoreMesh(core_axis_name="c", subcore_axis_name="s")

    @functools.partial(
        pl.kernel,
        mesh=mesh,
        out_type=[
            jax.ShapeDtypeStruct((NC * NP,), jnp.float32),
            jax.ShapeDtypeStruct((NC * NP,), jnp.float32),
            jax.ShapeDtypeStruct((NC * NP,), jnp.float32),
        ],
        scratch_types=[
            pltpu.VMEM((NCH2, CH), jnp.int32),
            pltpu.VMEM((ES2,), jnp.float32),
            pltpu.VMEM((ES2,), jnp.float32),
            pltpu.VMEM((ES2,), jnp.float32),
            pltpu.VMEM((RPT,), jnp.float32),
            pltpu.VMEM_SHARED((NP,), jnp.float32),
            pltpu.VMEM_SHARED((NP,), jnp.float32),
            pltpu.VMEM_SHARED((NP,), jnp.float32),
        ],
    )
    def k(tx_h, ty_h, tz_h, ix_h, px_h, py_h, pz_h,
          ix_v, txv, tyv, tzv, avbuf, accx, accy, accz):
        cid = lax.axis_index("c")
        sid = lax.axis_index("s")
        wid = cid * NS + sid
        pltpu.sync_copy(ix_h.at[wid], ix_v)
        eb = wid * ES2
        pltpu.sync_copy(tx_h.at[pl.ds(eb, ES2)], txv)
        pltpu.sync_copy(ty_h.at[pl.ds(eb, ES2)], tyv)
        pltpu.sync_copy(tz_h.at[pl.ds(eb, ES2)], tzv)

        def zb(i, carry):
            avbuf[pl.ds(i * 16, 16)] = jnp.zeros((16,), jnp.float32)
            return carry

        lax.fori_loop(0, RPT // 16, zb, 0)
        row0 = sid * RPT
        pltpu.sync_copy(avbuf, accx.at[pl.ds(row0, RPT)])
        pltpu.sync_copy(avbuf, accy.at[pl.ds(row0, RPT)])
        pltpu.sync_copy(avbuf, accz.at[pl.ds(row0, RPT)])
        plsc.subcore_barrier()

        def body(j, carry):
            sl = pl.ds(j * CH, CH)
            ixr = ix_v.at[j]
            pltpu.sync_copy(txv.at[sl], accx.at[ixr], add=True)
            pltpu.sync_copy(tyv.at[sl], accy.at[ixr], add=True)
            pltpu.sync_copy(tzv.at[sl], accz.at[ixr], add=True)
            return carry

        lax.fori_loop(0, NCH2, body, 0)
        plsc.subcore_barrier()

        pltpu.sync_copy(accx.at[pl.ds(row0, RPT)],
                        px_h.at[pl.ds(cid * NP + row0, RPT)])
        pltpu.sync_copy(accy.at[pl.ds(row0, RPT)],
                        py_h.at[pl.ds(cid * NP + row0, RPT)])
        pltpu.sync_copy(accz.at[pl.ds(row0, RPT)],
                        pz_h.at[pl.ds(cid * NP + row0, RPT)])

    return k(tx1, ty1, tz1, idx3s)


def _final(coord, px, py, pz):
    def body(c_ref, px_ref, py_ref, pz_ref, o_ref):
        ax = (px_ref[0] + px_ref[1]).reshape(N, 1)
        ay = (py_ref[0] + py_ref[1]).reshape(N, 1)
        az = (pz_ref[0] + pz_ref[1]).reshape(N, 1)
        agg = jnp.concatenate([ax, ay, az], axis=1)
        o_ref[...] = c_ref[...] + agg * (1.0 / NORM)

    return pl.pallas_call(
        body,
        grid=(1,),
        in_specs=[
            pl.BlockSpec((N, 3), lambda i: (0, 0)),
            pl.BlockSpec((NC, N), lambda i: (0, 0)),
            pl.BlockSpec((NC, N), lambda i: (0, 0)),
            pl.BlockSpec((NC, N), lambda i: (0, 0)),
        ],
        out_specs=pl.BlockSpec((N, 3), lambda i: (0, 0)),
        out_shape=jax.ShapeDtypeStruct((N, 3), jnp.float32),
    )(coord, px, py, pz)


def kernel(h, coord, edge_index, coord_diff, edge_attr, W1, b1, W2, b2, W3):
    f32 = jnp.float32
    row = edge_index[0].astype(jnp.int32)
    col = edge_index[1].astype(jnp.int32)

    W1a = W1[:H]
    W1b = W1[H:2 * H]
    w1c = W1[2 * H:2 * H + 1]
    I512 = jnp.eye(512, dtype=f32)

    A, B = _pre_node(h, W1a, W1b)

    idx4r = row.reshape(NS, 2, NCHS // 2, CH)
    idx4c = col.reshape(NS, 2, NCHS // 2, CH)
    Pg, Qg = _sc_gather(A, B, idx4r, idx4c)

    tx3, ty3, tz3 = _edge_mlp(
        Pg, Qg, coord_diff, edge_attr,
        w1c, b1.reshape(1, H), W2, b2.reshape(1, H), W3, I512)

    idx3s = row.reshape(NC * NS, NCH2, CH)
    px, py, pz = _sc_scatter(tx3.reshape(E), ty3.reshape(E), tz3.reshape(E),
                             idx3s)
    return _final(coord, px.reshape(NC, NP)[:, :N],
                  py.reshape(NC, NP)[:, :N], pz.reshape(NC, NP)[:, :N])


# speedup vs baseline: 1.1688x; 1.0953x over previous
import functools

import jax
import jax.numpy as jnp
from jax import lax
from jax.experimental import pallas as pl
from jax.experimental.pallas import tpu as pltpu
from jax.experimental.pallas import tpu_sc as plsc

N = 10000
E = 320000
H = 128
NORM = 100.0

NC = 2
NS = 16
CH = 80

BE = 12800
RB = BE // 512
GRID = E // BE
GROWS = E // 512

ES = E // NS
NCHS = ES // CH
ES2 = E // 32
NCH2 = ES2 // CH
NP = 10240
RPT = 640


def _pre_node(h, W1a, W1b):
    def body(h_ref, wa_ref, wb_ref, a_ref, b_ref):
        hv = h_ref[...]
        a_ref[...] = jnp.dot(hv, wa_ref[...], preferred_element_type=jnp.float32)
        b_ref[...] = jnp.dot(hv, wb_ref[...], preferred_element_type=jnp.float32)

    BN = 2000
    return pl.pallas_call(
        body,
        grid=(N // BN,),
        in_specs=[
            pl.BlockSpec((BN, H), lambda i: (i, 0)),
            pl.BlockSpec((H, H), lambda i: (0, 0)),
            pl.BlockSpec((H, H), lambda i: (0, 0)),
        ],
        out_specs=[
            pl.BlockSpec((BN, H), lambda i: (i, 0)),
            pl.BlockSpec((BN, H), lambda i: (i, 0)),
        ],
        out_shape=[
            jax.ShapeDtypeStruct((N, H), jnp.float32),
            jax.ShapeDtypeStruct((N, H), jnp.float32),
        ],
    )(h, W1a, W1b)


def _sc_gather(A, B, idx4r, idx4c):
    mesh = plsc.VectorSubcoreMesh(core_axis_name="c", subcore_axis_name="s")
    NHF = NCHS // 2

    @functools.partial(
        pl.kernel,
        mesh=mesh,
        out_type=[
            jax.ShapeDtypeStruct((E, H), jnp.float32),
            jax.ShapeDtypeStruct((E, H), jnp.float32),
        ],
        scratch_types=[
            pltpu.VMEM((NHF, CH), jnp.int32),
            pltpu.VMEM((CH, H), jnp.float32),
            pltpu.VMEM((CH, H), jnp.float32),
            pltpu.VMEM_SHARED((N, H), jnp.float32),
            pltpu.SemaphoreType.DMA,
            pltpu.SemaphoreType.DMA,
        ],
    )
    def k(a_hbm, b_hbm, ir_hbm, ic_hbm, p_hbm, q_hbm, ix_v, buf0, buf1,
          table, sem0, sem1):
        cid = lax.axis_index("c")
        sid = lax.axis_index("s")

        def run(t_hbm, i_hbm, o_hbm):
            row0 = sid * RPT

            @pl.when(sid < NS - 1)
            def _():
                pltpu.sync_copy(t_hbm.at[pl.ds(row0, RPT)],
                                table.at[pl.ds(row0, RPT)])

            @pl.when(sid == NS - 1)
            def _():
                last = N - (NS - 1) * RPT
                pltpu.sync_copy(t_hbm.at[pl.ds(row0, last)],
                                table.at[pl.ds(row0, last)])

            plsc.subcore_barrier()

            base = sid * ES
            bufs = (buf0, buf1)
            sems = (sem0, sem1)

            for hf in range(2):
                pltpu.sync_copy(i_hbm.at[sid, hf], ix_v)
                cb = hf * NHF
                pltpu.async_copy(table.at[ix_v.at[0]], buf0, sem0)

                def body(jj, carry):
                    for k2 in range(2):
                        j = jj * 2 + k2
                        cur = k2
                        nxt = 1 - k2
                        jn = j + 1

                        @pl.when(jn < NHF)
                        def _():
                            pltpu.async_copy(table.at[ix_v.at[jn]],
                                             bufs[nxt], sems[nxt])

                        off = base + (cb + j) * CH
                        pltpu.make_async_copy(table.at[ix_v.at[j]],
                                              bufs[cur], sems[cur]).wait()
                        pltpu.sync_copy(bufs[cur], o_hbm.at[pl.ds(off, CH)])
                    return carry

                lax.fori_loop(0, NHF // 2, body, 0)
                jl = NHF - 1
                off = base + (cb + jl) * CH
                pltpu.make_async_copy(table.at[ix_v.at[jl]], bufs[jl % 2],
                                      sems[jl % 2]).wait()
                pltpu.sync_copy(bufs[jl % 2], o_hbm.at[pl.ds(off, CH)])

        @pl.when(cid == 0)
        def _():
            run(a_hbm, ir_hbm, p_hbm)

        @pl.when(cid == 1)
        def _():
            run(b_hbm, ic_hbm, q_hbm)

    return k(A, B, idx4r, idx4c)


def _edge_mlp(P, Q, cd, ea, w1c, b1r, W2, b2r, W3, I512):
    def body(p_ref, q_ref, cd_ref, ea_ref, w1c_ref,
             b1_ref, w2_ref, b2_ref, w3_ref, i_ref,
             tx_ref, ty_ref, tz_ref):
        ident = i_ref[...]
        eaw = jnp.dot(ea_ref[...], w1c_ref[...],
                      preferred_element_type=jnp.float32)

        s = p_ref[...] + q_ref[...] + eaw + b1_ref[...]
        x1 = jax.nn.silu(s)
        y = (jnp.dot(x1, w2_ref[...], preferred_element_type=jnp.float32)
             + b2_ref[...])
        x2 = jax.nn.silu(y)
        m_col = jnp.dot(x2, w3_ref[...], preferred_element_type=jnp.float32)

        dn_row = (((0,), (0,)), ((), ()))
        m_rows = jnp.concatenate(
            [lax.dot_general(m_col[r * 512:(r + 1) * 512, :], ident, dn_row,
                             preferred_element_type=jnp.float32)
             for r in range(RB)], axis=0)

        cd_full = cd_ref[...]
        cds = [lax.dot_general(cd_full[r * 512:(r + 1) * 512, :], ident,
                               dn_row, preferred_element_type=jnp.float32)
               for r in range(RB)]
        cx_rows = jnp.concatenate([c[0:1] for c in cds], axis=0)
        cy_rows = jnp.concatenate([c[1:2] for c in cds], axis=0)
        cz_rows = jnp.concatenate([c[2:3] for c in cds], axis=0)

        tx_ref[...] = (cx_rows * m_rows)[None]
        ty_ref[...] = (cy_rows * m_rows)[None]
        tz_ref[...] = (cz_rows * m_rows)[None]

    rspec = pl.BlockSpec((1, RB, 512), lambda i: (i, 0, 0))
    rshape = jax.ShapeDtypeStruct((GRID, RB, 512), jnp.float32)
    return pl.pallas_call(
        body,
        grid=(GRID,),
        in_specs=[
            pl.BlockSpec((BE, H), lambda i: (i, 0)),
            pl.BlockSpec((BE, H), lambda i: (i, 0)),
            pl.BlockSpec((BE, 3), lambda i: (i, 0)),
            pl.BlockSpec((BE, 1), lambda i: (i, 0)),
            pl.BlockSpec((1, H), lambda i: (0, 0)),
            pl.BlockSpec((1, H), lambda i: (0, 0)),
            pl.BlockSpec((H, H), lambda i: (0, 0)),
            pl.BlockSpec((1, H), lambda i: (0, 0)),
            pl.BlockSpec((H, 1), lambda i: (0, 0)),
            pl.BlockSpec((512, 512), lambda i: (0, 0)),
        ],
        out_specs=[rspec, rspec, rspec],
        out_shape=[rshape, rshape, rshape],
    )(P, Q, cd, ea, w1c, b1r, W2, b2r, W3, I512)


def _sc_scatter(tx1, ty1, tz1, idx3s):
    mesh = plsc.VectorSubcoreMesh(core_axis_name="c", subcore_axis_name="s")

    @functools.partial(
        pl.kernel,
        mesh=mesh,
        out_type=[
            jax.ShapeDtypeStruct((NC * NP,), jnp.float32),
            jax.ShapeDtypeStruct((NC * NP,), jnp.float32),
            jax.ShapeDtypeStruct((NC * NP,), jnp.float32),
        ],
        scratch_types=[
            pltpu.VMEM((NCH2, CH), jnp.int32),
            pltpu.VMEM((ES2,), jnp.float32),
            pltpu.VMEM((ES2,), jnp.float32),
            pltpu.VMEM((ES2,), jnp.float32),
            pltpu.VMEM((RPT,), jnp.float32),
            pltpu.VMEM_SHARED((NP,), jnp.float32),
            pltpu.VMEM_SHARED((NP,), jnp.float32),
            pltpu.VMEM_SHARED((NP,), jnp.float32),
        ],
    )
    def k(tx_h, ty_h, tz_h, ix_h, px_h, py_h, pz_h,
          ix_v, txv, tyv, tzv, avbuf, accx, accy, accz):
        cid = lax.axis_index("c")
        sid = lax.axis_index("s")
        wid = cid * NS + sid
        pltpu.sync_copy(ix_h.at[wid], ix_v)
        eb = wid * ES2
        pltpu.sync_copy(tx_h.at[pl.ds(eb, ES2)], txv)
        pltpu.sync_copy(ty_h.at[pl.ds(eb, ES2)], tyv)
        pltpu.sync_copy(tz_h.at[pl.ds(eb, ES2)], tzv)

        def zb(i, carry):
            avbuf[pl.ds(i * 16, 16)] = jnp.zeros((16,), jnp.float32)
            return carry

        lax.fori_loop(0, RPT // 16, zb, 0)
        row0 = sid * RPT
        pltpu.sync_copy(avbuf, accx.at[pl.ds(row0, RPT)])
        pltpu.sync_copy(avbuf, accy.at[pl.ds(row0, RPT)])
        pltpu.sync_copy(avbuf, accz.at[pl.ds(row0, RPT)])
        plsc.subcore_barrier()

        def body(j, carry):
            sl = pl.ds(j * CH, CH)
            ixr = ix_v.at[j]
            pltpu.sync_copy(txv.at[sl], accx.at[ixr], add=True)
            pltpu.sync_copy(tyv.at[sl], accy.at[ixr], add=True)
            pltpu.sync_copy(tzv.at[sl], accz.at[ixr], add=True)
            return carry

        lax.fori_loop(0, NCH2, body, 0)
        plsc.subcore_barrier()

        pltpu.sync_copy(accx.at[pl.ds(row0, RPT)],
                        px_h.at[pl.ds(cid * NP + row0, RPT)])
        pltpu.sync_copy(accy.at[pl.ds(row0, RPT)],
                        py_h.at[pl.ds(cid * NP + row0, RPT)])
        pltpu.sync_copy(accz.at[pl.ds(row0, RPT)],
                        pz_h.at[pl.ds(cid * NP + row0, RPT)])

    return k(tx1, ty1, tz1, idx3s)


def _final(coord, px, py, pz):
    def body(c_ref, px_ref, py_ref, pz_ref, o_ref):
        ax = (px_ref[0] + px_ref[1]).reshape(N, 1)
        ay = (py_ref[0] + py_ref[1]).reshape(N, 1)
        az = (pz_ref[0] + pz_ref[1]).reshape(N, 1)
        agg = jnp.concatenate([ax, ay, az], axis=1)
        o_ref[...] = c_ref[...] + agg * (1.0 / NORM)

    return pl.pallas_call(
        body,
        grid=(1,),
        in_specs=[
            pl.BlockSpec((N, 3), lambda i: (0, 0)),
            pl.BlockSpec((NC, N), lambda i: (0, 0)),
            pl.BlockSpec((NC, N), lambda i: (0, 0)),
            pl.BlockSpec((NC, N), lambda i: (0, 0)),
        ],
        out_specs=pl.BlockSpec((N, 3), lambda i: (0, 0)),
        out_shape=jax.ShapeDtypeStruct((N, 3), jnp.float32),
    )(coord, px, py, pz)


def kernel(h, coord, edge_index, coord_diff, edge_attr, W1, b1, W2, b2, W3):
    f32 = jnp.float32
    row = edge_index[0].astype(jnp.int32)
    col = edge_index[1].astype(jnp.int32)

    W1a = W1[:H]
    W1b = W1[H:2 * H]
    w1c = W1[2 * H:2 * H + 1]
    I512 = jnp.eye(512, dtype=f32)

    A, B = _pre_node(h, W1a, W1b)

    idx4r = row.reshape(NS, 2, NCHS // 2, CH)
    idx4c = col.reshape(NS, 2, NCHS // 2, CH)
    Pg, Qg = _sc_gather(A, B, idx4r, idx4c)

    tx3, ty3, tz3 = _edge_mlp(
        Pg, Qg, coord_diff, edge_attr,
        w1c, b1.reshape(1, H), W2, b2.reshape(1, H), W3, I512)

    idx3s = row.reshape(NC * NS, NCH2, CH)
    px, py, pz = _sc_scatter(tx3.reshape(E), ty3.reshape(E), tz3.reshape(E),
                             idx3s)
    return _final(coord, px.reshape(NC, NP)[:, :N],
                  py.reshape(NC, NP)[:, :N], pz.reshape(NC, NP)[:, :N])
